# baseline (speedup 1.0000x reference)
import functools

import jax
import jax.numpy as jnp
from jax.experimental import pallas as pl

B = 2048
L = 10
IN = 300
INP = 384
D = 64
E = 8
SEL = 128
EPS = 1e-09

TS = 256


def _moe_kernel(x_ref, logits_ref, masks_ref, sel_ref, w_ref, b_ref,
                out_ref, guide_ref, selout_ref):
    step = pl.program_id(0)
    nsteps = pl.num_programs(0)

    logits = logits_ref[...]
    mask = (masks_ref[...] == 1).astype(jnp.float32)

    m = jnp.max(logits, axis=1, keepdims=True)
    ex = jnp.exp(logits - m)
    soft = ex / jnp.sum(ex, axis=1, keepdims=True)

    gated = soft * mask

    col = jax.lax.broadcasted_iota(jnp.int32, (TS, E), 1)
    m1 = jnp.max(gated, axis=1, keepdims=True)
    i1 = jnp.min(jnp.where(gated == m1, col, E), axis=1, keepdims=True)
    mask1 = col == i1
    gated2 = jnp.where(mask1, -1.0, gated)
    m2 = jnp.max(gated2, axis=1, keepdims=True)
    i2 = jnp.min(jnp.where(gated2 == m2, col, E), axis=1, keepdims=True)
    topk = mask1 | (col == i2)

    gatedk = gated * topk.astype(jnp.float32)
    gates = gatedk / (jnp.sum(gatedk, axis=1, keepdims=True) + EPS)

    inactive = soft * (1.0 - mask)
    inact = inactive / (jnp.sum(inactive, axis=1, keepdims=True) + EPS)
    e_i = jax.lax.broadcasted_iota(jnp.int32, (E, E * SEL), 0)
    j_i = jax.lax.broadcasted_iota(jnp.int32, (E, E * SEL), 1)
    s2 = (j_i // SEL == e_i).astype(jnp.bfloat16)
    ifull = jnp.dot(inact.astype(jnp.bfloat16), s2,
                    preferred_element_type=jnp.float32)
    sel = sel_ref[...]
    sel_acc = ifull[:, 0:SEL] * sel[:, 0, :]
    for e in range(1, E):
        sel_acc = sel_acc + ifull[:, e * SEL:(e + 1) * SEL] * sel[:, e, :]
    selout_ref[...] = sel_acc

    part = jnp.sum(soft * mask).reshape(1, 1)

    @pl.when(step == 0)
    def _init():
        guide_ref[...] = part

    @pl.when(step != 0)
    def _acc():
        guide_ref[...] = guide_ref[...] + part

    @pl.when(step == nsteps - 1)
    def _fin():
        s = guide_ref[...] / B
        guide_ref[...] = (1.0 - s) * (1.0 - s)

    gates_b = gates.astype(jnp.bfloat16)
    eg_i = jax.lax.broadcasted_iota(jnp.int32, (E, E * D), 0)
    jg_i = jax.lax.broadcasted_iota(jnp.int32, (E, E * D), 1)
    sg = (jg_i // D == eg_i).astype(jnp.bfloat16)
    gfull = jnp.dot(gates_b, sg, preferred_element_type=jnp.float32)
    gb = jnp.dot(gates_b, b_ref[...].astype(jnp.bfloat16),
                 preferred_element_type=jnp.float32)
    jr_i = jax.lax.broadcasted_iota(jnp.int32, (E * D, D), 0)
    orr_i = jax.lax.broadcasted_iota(jnp.int32, (E * D, D), 1)
    rg = (jr_i % D == orr_i).astype(jnp.bfloat16)

    x = x_ref[...]
    w = w_ref[...]
    for l in range(L):
        x_l = x[:, l * INP:(l + 1) * INP]
        y_l = jnp.dot(x_l, w, preferred_element_type=jnp.float32)
        z_l = (y_l * gfull).astype(jnp.bfloat16)
        o_l = jnp.dot(z_l, rg, preferred_element_type=jnp.float32) + gb
        out_ref[:, l, :] = o_l.astype(jnp.bfloat16)


@functools.partial(jax.jit, static_argnames=())
def kernel(cycle_curve_data, logits, moe_masks, selection_embeddings, W, b):
    xpad = jnp.pad(cycle_curve_data.astype(jnp.bfloat16),
                   ((0, 0), (0, 0), (0, INP - IN))).reshape(B, L * INP)
    wcat = jnp.pad(W.transpose(1, 0, 2).reshape(IN, E * D),
                   ((0, INP - IN), (0, 0))).astype(jnp.bfloat16)
    grid = (B // TS,)
    out, guide, selout = pl.pallas_call(
        _moe_kernel,
        grid=grid,
        in_specs=[
            pl.BlockSpec((TS, L * INP), lambda i: (i, 0)),
            pl.BlockSpec((TS, E), lambda i: (i, 0)),
            pl.BlockSpec((TS, E), lambda i: (i, 0)),
            pl.BlockSpec((TS, E, SEL), lambda i: (i, 0, 0)),
            pl.BlockSpec((INP, E * D), lambda i: (0, 0)),
            pl.BlockSpec((E, D), lambda i: (0, 0)),
        ],
        out_specs=[
            pl.BlockSpec((TS, L, D), lambda i: (i, 0, 0)),
            pl.BlockSpec((1, 1), lambda i: (0, 0)),
            pl.BlockSpec((TS, SEL), lambda i: (i, 0)),
        ],
        out_shape=[
            jax.ShapeDtypeStruct((B, L, D), jnp.bfloat16),
            jax.ShapeDtypeStruct((1, 1), jnp.float32),
            jax.ShapeDtypeStruct((B, SEL), jnp.float32),
        ],
    )(xpad, logits, moe_masks, selection_embeddings, wcat, b)
    return (out, guide[0, 0], selout)
